# speedup vs baseline: 6.1327x; 6.1327x over previous
import functools

import jax
import jax.numpy as jnp
from jax import lax
from jax.experimental import pallas as pl
from jax.experimental.pallas import tpu as pltpu

_HEAD_DIM = 128


def _mha_kernel(x_ref, w_ref, b_ref, o_ref, qkv_ref, *, n_heads, e):
    d = _HEAD_DIM
    x = x_ref[0]
    qkv_ref[...] = jnp.dot(
        x, w_ref[...], preferred_element_type=jnp.float32) + b_ref[...]

    for h in range(n_heads):
        q = qkv_ref[:, h * d:(h + 1) * d].astype(jnp.bfloat16)
        k = qkv_ref[:, e + h * d:e + (h + 1) * d].astype(jnp.bfloat16)
        v = qkv_ref[:, 2 * e + h * d:2 * e + (h + 1) * d].astype(jnp.bfloat16)
        s = lax.dot_general(q, k, (((1,), (1,)), ((), ())),
                            preferred_element_type=jnp.float32)
        m = jnp.max(s, axis=-1, keepdims=True)
        p = jnp.exp(s - m)
        l = jnp.sum(p, axis=-1, keepdims=True)
        acc = jnp.dot(p.astype(jnp.bfloat16), v,
                      preferred_element_type=jnp.float32)
        o_ref[0, :, h * d:(h + 1) * d] = (acc / l).astype(o_ref.dtype)


def kernel(x, w_qkv, b_qkv):
    B, S, E = x.shape
    n_heads = E // _HEAD_DIM
    xb = x.astype(jnp.bfloat16)
    wb = w_qkv.astype(jnp.bfloat16)
    return pl.pallas_call(
        functools.partial(_mha_kernel, n_heads=n_heads, e=E),
        out_shape=jax.ShapeDtypeStruct((B, S, E), x.dtype),
        grid=(B,),
        in_specs=[
            pl.BlockSpec((1, S, E), lambda b: (b, 0, 0)),
            pl.BlockSpec((E, 3 * E), lambda b: (0, 0)),
            pl.BlockSpec((1, 3 * E), lambda b: (0, 0)),
        ],
        out_specs=pl.BlockSpec((1, S, E), lambda b: (b, 0, 0)),
        scratch_shapes=[pltpu.VMEM((S, 3 * E), jnp.float32)],
        compiler_params=pltpu.CompilerParams(
            dimension_semantics=("parallel",)),
    )(xb, wb, b_qkv)


# speedup vs baseline: 7.0233x; 1.1452x over previous
import functools

import jax
import jax.numpy as jnp
from jax import lax
from jax.experimental import pallas as pl
from jax.experimental.pallas import tpu as pltpu

_HEAD_DIM = 128


def _mha_kernel(x_ref, w_ref, b_ref, o_ref, qkv_ref, *, n_heads, e):
    d = _HEAD_DIM
    x = x_ref[0].astype(jnp.bfloat16)
    qkv_ref[...] = jnp.dot(
        x, w_ref[...], preferred_element_type=jnp.float32) + b_ref[...]

    for h in range(n_heads):
        q = qkv_ref[:, h * d:(h + 1) * d].astype(jnp.bfloat16)
        k = qkv_ref[:, e + h * d:e + (h + 1) * d].astype(jnp.bfloat16)
        v = qkv_ref[:, 2 * e + h * d:2 * e + (h + 1) * d].astype(jnp.bfloat16)
        s = lax.dot_general(q, k, (((1,), (1,)), ((), ())),
                            preferred_element_type=jnp.float32)
        m = jnp.max(s, axis=-1, keepdims=True)
        p = jnp.exp(s - m)
        l = jnp.sum(p, axis=-1, keepdims=True)
        acc = jnp.dot(p.astype(jnp.bfloat16), v,
                      preferred_element_type=jnp.float32)
        o_ref[0, :, h * d:(h + 1) * d] = (acc / l).astype(o_ref.dtype)


def kernel(x, w_qkv, b_qkv):
    B, S, E = x.shape
    n_heads = E // _HEAD_DIM
    wb = w_qkv.astype(jnp.bfloat16)
    return pl.pallas_call(
        functools.partial(_mha_kernel, n_heads=n_heads, e=E),
        out_shape=jax.ShapeDtypeStruct((B, S, E), x.dtype),
        grid=(B,),
        in_specs=[
            pl.BlockSpec((1, S, E), lambda b: (b, 0, 0)),
            pl.BlockSpec((E, 3 * E), lambda b: (0, 0)),
            pl.BlockSpec((1, 3 * E), lambda b: (0, 0)),
        ],
        out_specs=pl.BlockSpec((1, S, E), lambda b: (b, 0, 0)),
        scratch_shapes=[pltpu.VMEM((S, 3 * E), jnp.float32)],
        compiler_params=pltpu.CompilerParams(
            dimension_semantics=("parallel",)),
    )(x, wb, b_qkv)
